# baseline (speedup 1.0000x reference)
import functools

import jax
import jax.numpy as jnp
from jax import lax
from jax.experimental import pallas as pl
from jax.experimental.pallas import tpu as pltpu
from jax.experimental.pallas import tpu_sc as plsc

B = 64
L = 512
DIN = 512
E = 2048
H = 1024
K = 30
NROWS = B * K
MT = 240
NT = NROWS // MT
SPS = MT // K


def _prep_body(text_ref, eosflat_ref):
    t = text_ref[...]
    col = lax.broadcasted_iota(jnp.int32, (B, L), 1)
    mx = jnp.max(t, axis=1, keepdims=True)
    eos = jnp.min(jnp.where(t == mx, col, L), axis=1, keepdims=True)
    base = lax.broadcasted_iota(jnp.int32, (B, 1), 0) * L
    eosflat_ref[...] = eos + base


def _topk_body(eosflat_ref, text_ref, atten_ref, gidx_ref, li_ref, rows_vmem, sem):
    copies = [
        pltpu.make_async_copy(
            atten_ref.at[pl.ds(eosflat_ref[b, 0], 1)],
            rows_vmem.at[pl.ds(b, 1)],
            sem,
        )
        for b in range(B)
    ]
    for c in copies:
        c.start()
    for c in copies:
        c.wait()

    t = text_ref[...]
    col = lax.broadcasted_iota(jnp.int32, (B, L), 1)
    mx = jnp.max(t, axis=1, keepdims=True)
    eos = jnp.min(jnp.where(t == mx, col, L), axis=1, keepdims=True)
    maskf = (t != 0).astype(jnp.float32)
    lengths = jnp.sum(maskf, axis=1, keepdims=True) - 2.0
    li_ref[...] = jnp.clip(lengths.astype(jnp.int32), 1, B - 1)

    row = rows_vmem[...]
    row = jnp.where(col == eos, -1.0, row)
    row = jnp.where(col == 0, -1.0, row)
    row = row * maskf

    base = lax.broadcasted_iota(jnp.int32, (B, 1), 0) * L
    colk = lax.broadcasted_iota(jnp.int32, (B, K), 1)
    acc = jnp.zeros((B, K), jnp.int32)
    neg_inf = jnp.float32(-jnp.inf)
    for j in range(K):
        m = jnp.max(row, axis=1, keepdims=True)
        pos = jnp.min(jnp.where(row == m, col, L), axis=1, keepdims=True)
        acc = jnp.where(colk == j, pos + base, acc)
        row = jnp.where(col == pos, neg_inf, row)
    gidx_ref[...] = acc


def _sc_gather(table2d, idx):
    info = plsc.get_sparse_core_info()
    rows_per = 64
    nw = NROWS // rows_per
    mesh = plsc.VectorSubcoreMesh(core_axis_name="c", subcore_axis_name="s")

    @functools.partial(
        pl.kernel,
        mesh=mesh,
        out_type=jax.ShapeDtypeStruct((NROWS, DIN), jnp.float32),
        scratch_types=[
            pltpu.VMEM((rows_per,), jnp.int32),
            pltpu.VMEM((rows_per, DIN), jnp.float32),
            pltpu.SemaphoreType.DMA,
        ],
    )
    def k(table_hbm, idx_hbm, out_hbm, idx_v, rows_v, sem):
        wid = lax.axis_index("s") * info.num_cores + lax.axis_index("c")

        @pl.when(wid < nw)
        def _():
            base = wid * rows_per
            pltpu.sync_copy(idx_hbm.at[pl.ds(base, rows_per)], idx_v)
            pltpu.async_copy(table_hbm.at[idx_v], rows_v, sem).wait()
            pltpu.sync_copy(rows_v, out_hbm.at[pl.ds(base, rows_per)])

    return k(table2d, idx)


def _dense_body(g_ref, w0_ref, b0_ref, gamma_ref, beta_ref, w1_ref, b1_ref,
                wd_ref, bd_ref, wlbig_ref, blin_ref, li_ref, out_ref,
                h_s, stats_s):
    t = pl.program_id(0)
    nreal = jnp.float32(NROWS)

    @pl.when(t < NT)
    def _():
        g = g_ref[...]
        nrm = jnp.sqrt(jnp.sum(g * g, axis=1, keepdims=True)) + 1e-8
        feats = (g / nrm).astype(jnp.bfloat16)
        h = lax.dot_general(feats, w0_ref[...], (((1,), (1,)), ((), ())),
                            preferred_element_type=jnp.float32) + b0_ref[...]
        h_s[pl.ds(t * MT, MT), :] = h
        s1 = jnp.sum(h, axis=0, keepdims=True)
        s2 = jnp.sum(h * h, axis=0, keepdims=True)
        contrib = jnp.concatenate([s1, s2], axis=0)

        @pl.when(t == 0)
        def _():
            stats_s[...] = contrib

        @pl.when(t != 0)
        def _():
            stats_s[...] = stats_s[...] + contrib

    @pl.when(t >= NT)
    def _():
        stats = stats_s[...]
        mu = stats[0:1, :] / nreal
        ex2 = stats[1:2, :] / nreal
        var = ex2 - mu * mu
        h = h_s[pl.ds((t - NT) * MT, MT), :]
        hn = (h - mu) / jnp.sqrt(var + 1e-5) * gamma_ref[...] + beta_ref[...]
        hn = jnp.maximum(hn, 0.0).astype(jnp.bfloat16)
        h2 = lax.dot_general(hn, w1_ref[...], (((1,), (1,)), ((), ())),
                             preferred_element_type=jnp.float32) + b1_ref[...]

        li = li_ref[...]
        h2r = h2.reshape(SPS, K, E)
        kio = lax.broadcasted_iota(jnp.int32, (SPS, K, 1), 1)
        valid3 = kio < li.reshape(SPS, 1, 1)
        neg_inf = jnp.float32(-jnp.inf)
        pooled = jnp.max(jnp.where(valid3, h2r, neg_inf), axis=1)

        g = g_ref[...]
        x1 = jnp.sum(g * wd_ref[...], axis=1, keepdims=True) + bd_ref[0, 0]
        contrib = x1 * wlbig_ref[...]
        rows = jnp.sum(contrib.reshape(SPS, K, E), axis=1) + blin_ref[...]
        nrm = jnp.sqrt(jnp.sum(rows * rows, axis=1, keepdims=True)) + 1e-8
        out_ref[...] = pooled + rows / nrm


def kernel(features, text, atten, pid, w_mlp0, b_mlp0, bn0_gamma, bn0_beta,
           w_mlp1, b_mlp1, w_dyn1, b_dyn1, w_lin1, b_lin1):
    atten2d = atten.reshape(B * L, L)
    features2d = features.reshape(B * L, DIN)

    eosflat = pl.pallas_call(
        _prep_body,
        out_shape=jax.ShapeDtypeStruct((B, 1), jnp.int32),
    )(text)

    gidx, li = pl.pallas_call(
        _topk_body,
        in_specs=[
            pl.BlockSpec(memory_space=pltpu.SMEM),
            pl.BlockSpec(memory_space=pltpu.VMEM),
            pl.BlockSpec(memory_space=pl.ANY),
        ],
        out_specs=[
            pl.BlockSpec(memory_space=pltpu.VMEM),
            pl.BlockSpec(memory_space=pltpu.VMEM),
        ],
        out_shape=[
            jax.ShapeDtypeStruct((B, K), jnp.int32),
            jax.ShapeDtypeStruct((B, 1), jnp.int32),
        ],
        scratch_shapes=[
            pltpu.VMEM((B, L), jnp.float32),
            pltpu.SemaphoreType.DMA,
        ],
    )(eosflat, text, atten2d)

    gathered = _sc_gather(features2d, gidx.reshape(NROWS))

    w0b = w_mlp0.astype(jnp.bfloat16)
    w1b = w_mlp1.astype(jnp.bfloat16)
    wlbig = jnp.tile(w_lin1.T, (SPS, 1))

    out = pl.pallas_call(
        _dense_body,
        grid=(2 * NT,),
        in_specs=[
            pl.BlockSpec((MT, DIN), lambda t: (t % NT, 0)),
            pl.BlockSpec((H, DIN), lambda t: (0, 0)),
            pl.BlockSpec((1, H), lambda t: (0, 0)),
            pl.BlockSpec((1, H), lambda t: (0, 0)),
            pl.BlockSpec((1, H), lambda t: (0, 0)),
            pl.BlockSpec((E, H), lambda t: (0, 0)),
            pl.BlockSpec((1, E), lambda t: (0, 0)),
            pl.BlockSpec((1, DIN), lambda t: (0, 0)),
            pl.BlockSpec(memory_space=pltpu.SMEM),
            pl.BlockSpec((MT, E), lambda t: (0, 0)),
            pl.BlockSpec((1, E), lambda t: (0, 0)),
            pl.BlockSpec((SPS, 1), lambda t: (t % NT, 0)),
        ],
        out_specs=pl.BlockSpec((SPS, E), lambda t: (jnp.maximum(t - NT, 0), 0)),
        out_shape=jax.ShapeDtypeStruct((B, E), jnp.float32),
        scratch_shapes=[
            pltpu.VMEM((NROWS, H), jnp.float32),
            pltpu.VMEM((2, H), jnp.float32),
        ],
    )(gathered, w0b, b_mlp0.reshape(1, H), bn0_gamma.reshape(1, H),
      bn0_beta.reshape(1, H), w1b, b_mlp1.reshape(1, E), w_dyn1,
      b_dyn1.reshape(1, 1), wlbig, b_lin1.reshape(1, E), li)

    return out


# speedup vs baseline: 1.0059x; 1.0059x over previous
import functools

import jax
import jax.numpy as jnp
from jax import lax
from jax.experimental import pallas as pl
from jax.experimental.pallas import tpu as pltpu
from jax.experimental.pallas import tpu_sc as plsc

B = 64
L = 512
DIN = 512
E = 2048
H = 1024
K = 30
KP = 32
NROWS = B * KP
MT = 256
NT = NROWS // MT
SPS = MT // KP


def _prep_body(text_ref, eosflat_ref):
    t = text_ref[...]
    col = lax.broadcasted_iota(jnp.int32, (B, L), 1)
    mx = jnp.max(t, axis=1, keepdims=True)
    eos = jnp.min(jnp.where(t == mx, col, L), axis=1, keepdims=True)
    base = lax.broadcasted_iota(jnp.int32, (B, 1), 0) * L
    eosflat_ref[...] = eos + base


def _topk_body(eosflat_ref, text_ref, atten_ref, gidx_ref, li_ref, rows_vmem, sem):
    copies = [
        pltpu.make_async_copy(
            atten_ref.at[pl.ds(eosflat_ref[b, 0], 1)],
            rows_vmem.at[pl.ds(b, 1)],
            sem,
        )
        for b in range(B)
    ]
    for c in copies:
        c.start()
    for c in copies:
        c.wait()

    t = text_ref[...]
    col = lax.broadcasted_iota(jnp.int32, (B, L), 1)
    mx = jnp.max(t, axis=1, keepdims=True)
    eos = jnp.min(jnp.where(t == mx, col, L), axis=1, keepdims=True)
    maskf = (t != 0).astype(jnp.float32)
    lengths = jnp.sum(maskf, axis=1, keepdims=True) - 2.0
    li_ref[...] = jnp.clip(lengths.astype(jnp.int32), 1, B - 1)

    row = rows_vmem[...]
    row = jnp.where(col == eos, -1.0, row)
    row = jnp.where(col == 0, -1.0, row)
    row = row * maskf

    base = lax.broadcasted_iota(jnp.int32, (B, 1), 0) * L
    colk = lax.broadcasted_iota(jnp.int32, (B, KP), 1)
    acc = jnp.zeros((B, KP), jnp.int32)
    neg_inf = jnp.float32(-jnp.inf)
    for j in range(K):
        m = jnp.max(row, axis=1, keepdims=True)
        pos = jnp.min(jnp.where(row == m, col, L), axis=1, keepdims=True)
        acc = jnp.where(colk == j, pos + base, acc)
        row = jnp.where(col == pos, neg_inf, row)
    gidx_ref[...] = acc


def _sc_gather(table2d, idx):
    info = plsc.get_sparse_core_info()
    nw = info.num_cores * info.num_subcores
    rows_per = NROWS // nw
    mesh = plsc.VectorSubcoreMesh(core_axis_name="c", subcore_axis_name="s")

    @functools.partial(
        pl.kernel,
        mesh=mesh,
        out_type=jax.ShapeDtypeStruct((NROWS, DIN), jnp.float32),
        scratch_types=[
            pltpu.VMEM((rows_per,), jnp.int32),
            pltpu.VMEM((rows_per, DIN), jnp.float32),
            pltpu.SemaphoreType.DMA,
        ],
    )
    def k(table_hbm, idx_hbm, out_hbm, idx_v, rows_v, sem):
        wid = lax.axis_index("s") * info.num_cores + lax.axis_index("c")
        base = wid * rows_per
        pltpu.sync_copy(idx_hbm.at[pl.ds(base, rows_per)], idx_v)
        pltpu.async_copy(table_hbm.at[idx_v], rows_v, sem).wait()
        pltpu.sync_copy(rows_v, out_hbm.at[pl.ds(base, rows_per)])

    return k(table2d, idx)


def _dense_body(g_ref, w0_ref, b0_ref, gamma_ref, beta_ref, w1_ref, b1_ref,
                wd_ref, bd_ref, wlbig_ref, blin_ref, li_ref, out_ref,
                h_s, stats_s):
    t = pl.program_id(0)
    nreal = jnp.float32(B * K)

    @pl.when(t < NT)
    def _():
        g = g_ref[...]
        nrm = jnp.sqrt(jnp.sum(g * g, axis=1, keepdims=True)) + 1e-8
        feats = (g / nrm).astype(jnp.bfloat16)
        h = lax.dot_general(feats, w0_ref[...], (((1,), (1,)), ((), ())),
                            preferred_element_type=jnp.float32) + b0_ref[...]
        h_s[pl.ds(t * MT, MT), :] = h
        rid = lax.broadcasted_iota(jnp.int32, (MT, 1), 0)
        valid = ((rid % KP) < K).astype(jnp.float32)
        hv = h * valid
        s1 = jnp.sum(hv, axis=0, keepdims=True)
        s2 = jnp.sum(hv * h, axis=0, keepdims=True)
        contrib = jnp.concatenate([s1, s2], axis=0)

        @pl.when(t == 0)
        def _():
            stats_s[...] = contrib

        @pl.when(t != 0)
        def _():
            stats_s[...] = stats_s[...] + contrib

    @pl.when(t >= NT)
    def _():
        stats = stats_s[...]
        mu = stats[0:1, :] / nreal
        ex2 = stats[1:2, :] / nreal
        var = ex2 - mu * mu
        h = h_s[pl.ds((t - NT) * MT, MT), :]
        hn = (h - mu) / jnp.sqrt(var + 1e-5) * gamma_ref[...] + beta_ref[...]
        hn = jnp.maximum(hn, 0.0).astype(jnp.bfloat16)
        h2 = lax.dot_general(hn, w1_ref[...], (((1,), (1,)), ((), ())),
                             preferred_element_type=jnp.float32) + b1_ref[...]

        li = jnp.minimum(li_ref[...], K)
        h2r = h2.reshape(SPS, KP, E)
        kio = lax.broadcasted_iota(jnp.int32, (SPS, KP, 1), 1)
        valid3 = kio < li.reshape(SPS, 1, 1)
        neg_inf = jnp.float32(-jnp.inf)
        pooled = jnp.max(jnp.where(valid3, h2r, neg_inf), axis=1)

        g = g_ref[...]
        x1 = jnp.sum(g * wd_ref[...], axis=1, keepdims=True) + bd_ref[0, 0]
        contrib = x1 * wlbig_ref[...]
        rows = jnp.sum(contrib.reshape(SPS, KP, E), axis=1) + blin_ref[...]
        nrm = jnp.sqrt(jnp.sum(rows * rows, axis=1, keepdims=True)) + 1e-8
        out_ref[...] = pooled + rows / nrm


def kernel(features, text, atten, pid, w_mlp0, b_mlp0, bn0_gamma, bn0_beta,
           w_mlp1, b_mlp1, w_dyn1, b_dyn1, w_lin1, b_lin1):
    atten2d = atten.reshape(B * L, L)
    features2d = features.reshape(B * L, DIN)

    eosflat = pl.pallas_call(
        _prep_body,
        out_shape=jax.ShapeDtypeStruct((B, 1), jnp.int32),
    )(text)

    gidx, li = pl.pallas_call(
        _topk_body,
        in_specs=[
            pl.BlockSpec(memory_space=pltpu.SMEM),
            pl.BlockSpec(memory_space=pltpu.VMEM),
            pl.BlockSpec(memory_space=pl.ANY),
        ],
        out_specs=[
            pl.BlockSpec(memory_space=pltpu.VMEM),
            pl.BlockSpec(memory_space=pltpu.VMEM),
        ],
        out_shape=[
            jax.ShapeDtypeStruct((B, KP), jnp.int32),
            jax.ShapeDtypeStruct((B, 1), jnp.int32),
        ],
        scratch_shapes=[
            pltpu.VMEM((B, L), jnp.float32),
            pltpu.SemaphoreType.DMA,
        ],
    )(eosflat, text, atten2d)

    gathered = _sc_gather(features2d, gidx.reshape(NROWS))

    w0b = w_mlp0.astype(jnp.bfloat16)
    w1b = w_mlp1.astype(jnp.bfloat16)
    wlbig = jnp.tile(jnp.pad(w_lin1, ((0, 0), (0, KP - K))).T, (SPS, 1))

    out = pl.pallas_call(
        _dense_body,
        grid=(2 * NT,),
        in_specs=[
            pl.BlockSpec((MT, DIN), lambda t: (t % NT, 0)),
            pl.BlockSpec((H, DIN), lambda t: (0, 0)),
            pl.BlockSpec((1, H), lambda t: (0, 0)),
            pl.BlockSpec((1, H), lambda t: (0, 0)),
            pl.BlockSpec((1, H), lambda t: (0, 0)),
            pl.BlockSpec((E, H), lambda t: (0, 0)),
            pl.BlockSpec((1, E), lambda t: (0, 0)),
            pl.BlockSpec((1, DIN), lambda t: (0, 0)),
            pl.BlockSpec(memory_space=pltpu.SMEM),
            pl.BlockSpec((MT, E), lambda t: (0, 0)),
            pl.BlockSpec((1, E), lambda t: (0, 0)),
            pl.BlockSpec((SPS, 1), lambda t: (t % NT, 0)),
        ],
        out_specs=pl.BlockSpec((SPS, E), lambda t: (jnp.maximum(t - NT, 0), 0)),
        out_shape=jax.ShapeDtypeStruct((B, E), jnp.float32),
        scratch_shapes=[
            pltpu.VMEM((NROWS, H), jnp.float32),
            pltpu.VMEM((2, H), jnp.float32),
        ],
    )(gathered, w0b, b_mlp0.reshape(1, H), bn0_gamma.reshape(1, H),
      bn0_beta.reshape(1, H), w1b, b_mlp1.reshape(1, E), w_dyn1,
      b_dyn1.reshape(1, 1), wlbig, b_lin1.reshape(1, E), li)

    return out


# speedup vs baseline: 1.0871x; 1.0807x over previous
import functools

import jax
import jax.numpy as jnp
from jax import lax
from jax.experimental import pallas as pl
from jax.experimental.pallas import tpu as pltpu
from jax.experimental.pallas import tpu_sc as plsc

B = 64
L = 512
DIN = 512
E = 2048
H = 1024
K = 30
KP = 32
NROWS = B * KP
MT = 512
NT = NROWS // MT
SPS = MT // KP


def _prep_body(text_ref, eosflat_ref):
    t = text_ref[...]
    col = lax.broadcasted_iota(jnp.int32, (B, L), 1)
    mx = jnp.max(t, axis=1, keepdims=True)
    eos = jnp.min(jnp.where(t == mx, col, L), axis=1, keepdims=True)
    base = lax.broadcasted_iota(jnp.int32, (B, 1), 0) * L
    eosflat_ref[...] = eos + base


def _topk_body(eosflat_ref, text_ref, atten_ref, gidx_ref, li_ref, rows_vmem, sem):
    copies = [
        pltpu.make_async_copy(
            atten_ref.at[pl.ds(eosflat_ref[b, 0], 1)],
            rows_vmem.at[pl.ds(b, 1)],
            sem,
        )
        for b in range(B)
    ]
    for c in copies:
        c.start()
    for c in copies:
        c.wait()

    t = text_ref[...]
    col = lax.broadcasted_iota(jnp.int32, (B, L), 1)
    mx = jnp.max(t, axis=1, keepdims=True)
    eos = jnp.min(jnp.where(t == mx, col, L), axis=1, keepdims=True)
    maskf = (t != 0).astype(jnp.float32)
    lengths = jnp.sum(maskf, axis=1, keepdims=True) - 2.0
    li_ref[...] = jnp.clip(lengths.astype(jnp.int32), 1, B - 1)

    row = rows_vmem[...]
    row = jnp.where(col == eos, -1.0, row)
    row = jnp.where(col == 0, -1.0, row)
    row = row * maskf

    base = lax.broadcasted_iota(jnp.int32, (B, 1), 0) * L
    colk = lax.broadcasted_iota(jnp.int32, (B, KP), 1)
    acc = jnp.zeros((B, KP), jnp.int32)
    neg_inf = jnp.float32(-jnp.inf)
    for j in range(K):
        m = jnp.max(row, axis=1, keepdims=True)
        pos = jnp.min(jnp.where(row == m, col, L), axis=1, keepdims=True)
        acc = jnp.where(colk == j, pos + base, acc)
        row = jnp.where(col == pos, neg_inf, row)
    gidx_ref[...] = acc


def _sc_gather(table2d, idx):
    info = plsc.get_sparse_core_info()
    nw = info.num_cores * info.num_subcores
    rows_per = NROWS // nw
    mesh = plsc.VectorSubcoreMesh(core_axis_name="c", subcore_axis_name="s")

    @functools.partial(
        pl.kernel,
        mesh=mesh,
        out_type=jax.ShapeDtypeStruct((NROWS, DIN), jnp.float32),
        scratch_types=[
            pltpu.VMEM((rows_per,), jnp.int32),
            pltpu.VMEM((rows_per, DIN), jnp.float32),
            pltpu.SemaphoreType.DMA,
        ],
    )
    def k(table_hbm, idx_hbm, out_hbm, idx_v, rows_v, sem):
        wid = lax.axis_index("s") * info.num_cores + lax.axis_index("c")
        base = wid * rows_per
        pltpu.sync_copy(idx_hbm.at[pl.ds(base, rows_per)], idx_v)
        pltpu.async_copy(table_hbm.at[idx_v], rows_v, sem).wait()
        pltpu.sync_copy(rows_v, out_hbm.at[pl.ds(base, rows_per)])

    return k(table2d, idx)


def _dense_body(g_ref, w0_ref, b0_ref, gamma_ref, beta_ref, w1_ref, b1_ref,
                wd_ref, bd_ref, wlbig_ref, blin_ref, li_ref, out_ref,
                h_s, stats_s, w0b_s, w1b_s):
    t = pl.program_id(0)
    nreal = jnp.float32(B * K)

    @pl.when(t == 0)
    def _():
        w0b_s[...] = w0_ref[...].astype(jnp.bfloat16)

    @pl.when(t == NT)
    def _():
        w1b_s[...] = w1_ref[...].astype(jnp.bfloat16)

    @pl.when(t < NT)
    def _():
        g = g_ref[...]
        nrm = jnp.sqrt(jnp.sum(g * g, axis=1, keepdims=True)) + 1e-8
        feats = (g / nrm).astype(jnp.bfloat16)
        h = lax.dot_general(feats, w0b_s[...], (((1,), (1,)), ((), ())),
                            preferred_element_type=jnp.float32) + b0_ref[...]
        h_s[pl.ds(t * MT, MT), :] = h
        rid = lax.broadcasted_iota(jnp.int32, (MT, 1), 0)
        valid = ((rid % KP) < K).astype(jnp.float32)
        hv = h * valid
        s1 = jnp.sum(hv, axis=0, keepdims=True)
        s2 = jnp.sum(hv * h, axis=0, keepdims=True)
        contrib = jnp.concatenate([s1, s2], axis=0)

        @pl.when(t == 0)
        def _():
            stats_s[...] = contrib

        @pl.when(t != 0)
        def _():
            stats_s[...] = stats_s[...] + contrib

    @pl.when(t >= NT)
    def _():
        stats = stats_s[...]
        mu = stats[0:1, :] / nreal
        ex2 = stats[1:2, :] / nreal
        var = ex2 - mu * mu
        h = h_s[pl.ds((t - NT) * MT, MT), :]
        hn = (h - mu) / jnp.sqrt(var + 1e-5) * gamma_ref[...] + beta_ref[...]
        hn = jnp.maximum(hn, 0.0).astype(jnp.bfloat16)
        h2 = lax.dot_general(hn, w1b_s[...], (((1,), (1,)), ((), ())),
                             preferred_element_type=jnp.float32) + b1_ref[...]

        li = jnp.minimum(li_ref[...], K)
        h2r = h2.reshape(SPS, KP, E)
        kio = lax.broadcasted_iota(jnp.int32, (SPS, KP, 1), 1)
        valid3 = kio < li.reshape(SPS, 1, 1)
        neg_inf = jnp.float32(-jnp.inf)
        pooled = jnp.max(jnp.where(valid3, h2r, neg_inf), axis=1)

        g = g_ref[...]
        x1 = jnp.sum(g * wd_ref[...], axis=1, keepdims=True) + bd_ref[0, 0]
        contrib = x1 * wlbig_ref[...]
        rows = jnp.sum(contrib.reshape(SPS, KP, E), axis=1) + blin_ref[...]
        nrm = jnp.sqrt(jnp.sum(rows * rows, axis=1, keepdims=True)) + 1e-8
        out_ref[...] = pooled + rows / nrm


def kernel(features, text, atten, pid, w_mlp0, b_mlp0, bn0_gamma, bn0_beta,
           w_mlp1, b_mlp1, w_dyn1, b_dyn1, w_lin1, b_lin1):
    atten2d = atten.reshape(B * L, L)
    features2d = features.reshape(B * L, DIN)

    eosflat = pl.pallas_call(
        _prep_body,
        out_shape=jax.ShapeDtypeStruct((B, 1), jnp.int32),
    )(text)

    gidx, li = pl.pallas_call(
        _topk_body,
        in_specs=[
            pl.BlockSpec(memory_space=pltpu.SMEM),
            pl.BlockSpec(memory_space=pltpu.VMEM),
            pl.BlockSpec(memory_space=pl.ANY),
        ],
        out_specs=[
            pl.BlockSpec(memory_space=pltpu.VMEM),
            pl.BlockSpec(memory_space=pltpu.VMEM),
        ],
        out_shape=[
            jax.ShapeDtypeStruct((B, KP), jnp.int32),
            jax.ShapeDtypeStruct((B, 1), jnp.int32),
        ],
        scratch_shapes=[
            pltpu.VMEM((B, L), jnp.float32),
            pltpu.SemaphoreType.DMA,
        ],
    )(eosflat, text, atten2d)

    gathered = _sc_gather(features2d, gidx.reshape(NROWS))

    wlbig = jnp.tile(jnp.pad(w_lin1, ((0, 0), (0, KP - K))).T, (SPS, 1))

    out = pl.pallas_call(
        _dense_body,
        grid=(2 * NT,),
        in_specs=[
            pl.BlockSpec((MT, DIN), lambda t: (t % NT, 0)),
            pl.BlockSpec((H, DIN), lambda t: (0, 0)),
            pl.BlockSpec((1, H), lambda t: (0, 0)),
            pl.BlockSpec((1, H), lambda t: (0, 0)),
            pl.BlockSpec((1, H), lambda t: (0, 0)),
            pl.BlockSpec((E, H), lambda t: (0, 0)),
            pl.BlockSpec((1, E), lambda t: (0, 0)),
            pl.BlockSpec((1, DIN), lambda t: (0, 0)),
            pl.BlockSpec(memory_space=pltpu.SMEM),
            pl.BlockSpec((MT, E), lambda t: (0, 0)),
            pl.BlockSpec((1, E), lambda t: (0, 0)),
            pl.BlockSpec((SPS, 1), lambda t: (t % NT, 0)),
        ],
        out_specs=pl.BlockSpec((SPS, E), lambda t: (jnp.maximum(t - NT, 0), 0)),
        out_shape=jax.ShapeDtypeStruct((B, E), jnp.float32),
        scratch_shapes=[
            pltpu.VMEM((NROWS, H), jnp.float32),
            pltpu.VMEM((2, H), jnp.float32),
            pltpu.VMEM((H, DIN), jnp.bfloat16),
            pltpu.VMEM((E, H), jnp.bfloat16),
        ],
    )(gathered, w_mlp0, b_mlp0.reshape(1, H), bn0_gamma.reshape(1, H),
      bn0_beta.reshape(1, H), w_mlp1, b_mlp1.reshape(1, E), w_dyn1,
      b_dyn1.reshape(1, 1), wlbig, b_lin1.reshape(1, E), li)

    return out
